# baseline (speedup 1.0000x reference)
import functools

import jax
import jax.numpy as jnp
from jax import lax
from jax.experimental import pallas as pl
from jax.experimental.pallas import tpu as pltpu
from jax.experimental.pallas import tpu_sc as plsc

_R = 4
_G = 64
_P = 0.1
_NC = 2
_NS = 16
_CH = 80
_WD = 16


def _mesh():
    return plsc.VectorSubcoreMesh(
        core_axis_name="c", subcore_axis_name="s",
        num_cores=_NC, num_subcores=_NS)


def _seg_sum_call(table, src, dst, zeros, R, N, d, E, nh):
    SUP = 8
    EPT = E // _NS
    NIT = EPT // (SUP * _CH)
    TAIL = EPT // _CH - NIT * SUP
    RPC = R // _NC
    RPT = (N // _NS) // 8 * 8
    RLAST = N - (_NS - 1) * RPT

    @functools.partial(
        pl.kernel,
        out_type=jax.ShapeDtypeStruct((nh, R * N, d), jnp.float32),
        mesh=_mesh(),
        scratch_types=[
            pltpu.VMEM_SHARED((N, d), jnp.float32),
            pltpu.VMEM_SHARED((N, d), jnp.float32),
            pltpu.VMEM((SUP, _CH), jnp.int32),
            pltpu.VMEM((SUP, _CH), jnp.int32),
            pltpu.VMEM((SUP, _CH, d), jnp.float32),
            pltpu.SemaphoreType.DMA,
            pltpu.SemaphoreType.DMA,
        ],
        compiler_params=pltpu.CompilerParams(use_tc_tiling_on_sc=False),
    )
    def k(t_hbm, src_hbm, dst_hbm, z_hbm, out_hbm, acc, hsp, sidx, didx,
          rows, gsem, ssem):
        c = lax.axis_index("c")
        s = lax.axis_index("s")
        drow0 = s * EPT // _CH
        for half in range(nh):
            for step in range(RPC):
                r = c * RPC + step

                @pl.when(s < _NS - 1)
                def _():
                    pltpu.sync_copy(
                        t_hbm.at[half, pl.ds(r * N + s * RPT, RPT)],
                        hsp.at[pl.ds(s * RPT, RPT)])
                    pltpu.sync_copy(z_hbm.at[pl.ds(0, RPT)],
                                    acc.at[pl.ds(s * RPT, RPT)])

                @pl.when(s == _NS - 1)
                def _():
                    pltpu.sync_copy(
                        t_hbm.at[half, pl.ds(r * N + (_NS - 1) * RPT, RLAST)],
                        hsp.at[pl.ds((_NS - 1) * RPT, RLAST)])
                    pltpu.sync_copy(z_hbm,
                                    acc.at[pl.ds((_NS - 1) * RPT, RLAST)])

                plsc.subcore_barrier()

                def process(drow, ns):
                    pltpu.sync_copy(src_hbm.at[pl.ds(drow, ns)],
                                    sidx.at[pl.ds(0, ns)])
                    pltpu.sync_copy(dst_hbm.at[pl.ds(drow, ns)],
                                    didx.at[pl.ds(0, ns)])
                    gd = [pltpu.async_copy(hsp.at[sidx.at[j]], rows.at[j],
                                           gsem)
                          for j in range(ns)]
                    for j in range(ns):
                        gd[j].wait()
                    sd = [pltpu.async_copy(rows.at[j], acc.at[didx.at[j]],
                                           ssem, add=True)
                          for j in range(ns)]
                    for j in range(ns):
                        sd[j].wait()

                def body(i, carry):
                    process(drow0 + i * SUP, SUP)
                    return carry

                lax.fori_loop(0, NIT, body, 0)
                if TAIL:
                    process(drow0 + NIT * SUP, TAIL)
                plsc.subcore_barrier()

                @pl.when(s < _NS - 1)
                def _():
                    pltpu.sync_copy(
                        acc.at[pl.ds(s * RPT, RPT)],
                        out_hbm.at[half, pl.ds(r * N + s * RPT, RPT)])

                @pl.when(s == _NS - 1)
                def _():
                    pltpu.sync_copy(
                        acc.at[pl.ds((_NS - 1) * RPT, RLAST)],
                        out_hbm.at[half,
                                   pl.ds(r * N + (_NS - 1) * RPT, RLAST)])

                plsc.subcore_barrier()

    return k(table, src, dst, zeros)


def _pool_sc_call(y, batch, zg, N, G):
    NCH = N // _CH
    NW = _NC * _NS
    PER = -(-NCH // NW)

    @functools.partial(
        pl.kernel,
        out_type=jax.ShapeDtypeStruct((_NC, G, _WD), jnp.float32),
        mesh=_mesh(),
        scratch_types=[
            pltpu.VMEM_SHARED((G, _WD), jnp.float32),
            pltpu.VMEM((_CH,), jnp.int32),
            pltpu.VMEM((_CH, _WD), jnp.float32),
        ],
        compiler_params=pltpu.CompilerParams(use_tc_tiling_on_sc=False),
    )
    def k(y_hbm, b_hbm, z_hbm, out_hbm, acc, bidx, rows):
        c = lax.axis_index("c")
        s = lax.axis_index("s")
        w = c * _NS + s

        @pl.when(s == 0)
        def _():
            pltpu.sync_copy(z_hbm, acc)

        plsc.subcore_barrier()
        for kk in range(PER):
            ch = kk * NW + w

            @pl.when(ch < NCH)
            def _():
                base = ch * _CH
                pltpu.sync_copy(b_hbm.at[pl.ds(base, _CH)], bidx)
                pltpu.sync_copy(y_hbm.at[pl.ds(base, _CH)], rows)
                pltpu.sync_copy(rows, acc.at[bidx], add=True)

        plsc.subcore_barrier()

        @pl.when(s == 0)
        def _():
            pltpu.sync_copy(acc, out_hbm.at[c])

    return k(y, batch, zg)


def _xr_call(x, keep3, R, N, F):
    BN = 400

    def body(x_ref, k_ref, o_ref):
        o_ref[...] = x_ref[...][None] * k_ref[...]

    return pl.pallas_call(
        body,
        grid=(R, N // BN),
        in_specs=[pl.BlockSpec((BN, F), lambda r, b: (b, 0)),
                  pl.BlockSpec((1, BN, 1), lambda r, b: (r, b, 0))],
        out_specs=pl.BlockSpec((1, BN, F), lambda r, b: (r, b, 0)),
        out_shape=jax.ShapeDtypeStruct((R, N, F), jnp.float32),
    )(x, keep3)


def _mlp_call(h, agg, nh, w1, b1, w2, b2, M, din, dout):
    BN = 400
    dh = din // nh

    def body(*refs):
        h_ref = refs[0]
        a_refs = refs[1:1 + nh]
        w1_ref, b1_ref, w2_ref, b2_ref, o_ref = refs[1 + nh:]
        if nh == 1:
            a = a_refs[0][0]
        else:
            a = jnp.concatenate([ar[0] for ar in a_refs], axis=-1)
        z = h_ref[...] + a
        z = jnp.dot(z, w1_ref[...], preferred_element_type=jnp.float32)
        z = jnp.maximum(z + b1_ref[...], 0.0)
        z = jnp.dot(z, w2_ref[...], preferred_element_type=jnp.float32)
        o_ref[...] = jnp.maximum(z + b2_ref[...], 0.0)

    in_specs = [pl.BlockSpec((BN, din), lambda b: (b, 0))]
    for half in range(nh):
        in_specs.append(
            pl.BlockSpec((1, BN, dh), lambda b, h=half: (h, b, 0)))
    in_specs += [pl.BlockSpec((din, dout), lambda b: (0, 0)),
                 pl.BlockSpec((1, dout), lambda b: (0, 0)),
                 pl.BlockSpec((dout, dout), lambda b: (0, 0)),
                 pl.BlockSpec((1, dout), lambda b: (0, 0))]
    return pl.pallas_call(
        body,
        grid=(M // BN,),
        in_specs=in_specs,
        out_specs=pl.BlockSpec((BN, dout), lambda b: (b, 0)),
        out_shape=jax.ShapeDtypeStruct((M, dout), jnp.float32),
    )(h, *([agg] * nh), w1, b1, w2, b2)


def _pernode_call(x, km, hs, w0, ws, N, F, D, R):
    BN = 400
    nh = len(hs)

    def body(*refs):
        x_ref, km_ref = refs[0], refs[1]
        h_refs = refs[2:2 + nh]
        w0_ref = refs[2 + nh]
        w_refs = refs[3 + nh:3 + 2 * nh]
        o_ref = refs[-1]
        y = jnp.dot(x_ref[...] * km_ref[...], w0_ref[...],
                    preferred_element_type=jnp.float32)
        for hr, wr in zip(h_refs, w_refs):
            m = hr[0]
            for r in range(1, R):
                m = m + hr[r]
            y = y + jnp.dot(m * (1.0 / R), wr[...],
                            preferred_element_type=jnp.float32)
        o_ref[...] = y

    in_specs = [pl.BlockSpec((BN, F), lambda b: (b, 0)),
                pl.BlockSpec((BN, 1), lambda b: (b, 0))]
    in_specs += [pl.BlockSpec((R, BN, D), lambda b: (0, b, 0))] * nh
    in_specs += [pl.BlockSpec((F, _WD), lambda b: (0, 0))]
    in_specs += [pl.BlockSpec((D, _WD), lambda b: (0, 0))] * nh
    return pl.pallas_call(
        body,
        grid=(N // BN,),
        in_specs=in_specs,
        out_specs=pl.BlockSpec((BN, _WD), lambda b: (b, 0)),
        out_shape=jax.ShapeDtypeStruct((N, _WD), jnp.float32),
    )(x, km, *hs, w0, *ws)


def _head_call(parts, bsum, G, C):
    def body(p_ref, b_ref, o_ref):
        p = p_ref[0] + p_ref[1] + b_ref[...]
        col = lax.broadcasted_iota(jnp.int32, (G, _WD), 1)
        valid = col < C
        pm = jnp.where(valid, p, jnp.full_like(p, -1e30))
        m = jnp.max(pm, axis=1, keepdims=True)
        e = jnp.where(valid, jnp.exp(p - m), 0.0)
        ssum = jnp.sum(e, axis=1, keepdims=True)
        o_ref[...] = p - m - jnp.log(ssum)

    return pl.pallas_call(
        body,
        in_specs=[pl.BlockSpec((2, G, _WD), lambda: (0, 0, 0)),
                  pl.BlockSpec((1, _WD), lambda: (0, 0))],
        out_specs=pl.BlockSpec((G, _WD), lambda: (0, 0)),
        out_shape=jax.ShapeDtypeStruct((G, _WD), jnp.float32),
    )(parts, bsum)


def kernel(x, edge_index, batch, params):
    N, F = x.shape
    E = edge_index.shape[1]
    D = params["conv0_w1"].shape[1]
    C = params["fc0_w"].shape[1]
    L = sum(1 for k in params if k.startswith("conv") and k.endswith("_w1"))
    R, G = _R, _G
    f32 = jnp.float32

    drop = jax.random.bernoulli(jax.random.key(42), _P, (R, N))
    keep = 1.0 - drop.astype(f32)
    keep3 = keep[:, :, None]
    km = jnp.mean(keep, axis=0)[:, None]

    src = edge_index[0].reshape(E // _CH, _CH)
    dst = edge_index[1].reshape(E // _CH, _CH)

    s_bn = 1.0 / jnp.sqrt(jnp.asarray(1.0 + 1e-5, f32))

    xr = _xr_call(x, keep3, R, N, F)
    h = xr.reshape(R * N, F)
    zeros = jnp.zeros((N - (_NS - 1) * ((N // _NS) // 8 * 8), D), f32)
    hs = []
    din = F
    for i in range(L):
        g1 = params[f"conv{i}_bn_g"] * s_bn
        w1 = params[f"conv{i}_w1"] * g1[None, :]
        b1 = (params[f"conv{i}_b1"] * g1 + params[f"conv{i}_bn_b"])[None, :]
        g2 = params[f"bn{i}_g"] * s_bn
        w2 = params[f"conv{i}_w2"] * g2[None, :]
        b2 = (params[f"conv{i}_b2"] * g2 + params[f"bn{i}_b"])[None, :]
        nh = din // D
        if nh == 1:
            table = h[None]
        else:
            table = (h.reshape(R * N, nh, D)
                     .transpose(1, 0, 2))
        agg = _seg_sum_call(table, src, dst, zeros, R, N, D, E, nh)
        h = _mlp_call(h, agg, nh, w1, b1, w2, b2, R * N, din, D)
        hs.append(h.reshape(R, N, D))
        din = D

    w0 = jnp.zeros((F, _WD), f32).at[:, :C].set(params["fc0_w"])
    ws = [jnp.zeros((D, _WD), f32).at[:, :C].set(params[f"fc{i + 1}_w"])
          for i in range(L)]
    bsum = sum(params[f"fc{i}_b"] for i in range(L + 1))
    bsum16 = jnp.zeros((1, _WD), f32).at[0, :C].set(bsum)

    y = _pernode_call(x, km, hs, w0, ws, N, F, D, R)
    zg = jnp.zeros((G, _WD), f32)
    parts = _pool_sc_call(y, batch, zg, N, G)
    out16 = _head_call(parts, bsum16, G, C)
    return out16[:, :C]


# speedup vs baseline: 1.0186x; 1.0186x over previous
import functools

import jax
import jax.numpy as jnp
from jax import lax
from jax.experimental import pallas as pl
from jax.experimental.pallas import tpu as pltpu
from jax.experimental.pallas import tpu_sc as plsc

_R = 4
_G = 64
_P = 0.1
_NC = 2
_NS = 16
_CH = 80
_WD = 16


def _mesh():
    return plsc.VectorSubcoreMesh(
        core_axis_name="c", subcore_axis_name="s",
        num_cores=_NC, num_subcores=_NS)


def _seg_sum_call(table, src, dst, zeros, R, N, d, E, nh):
    SUP = 8
    EPT = E // _NS
    NIT = EPT // (SUP * _CH)
    TAIL = EPT // _CH - NIT * SUP
    RPC = R // _NC
    RPT = (N // _NS) // 8 * 8
    RLAST = N - (_NS - 1) * RPT

    @functools.partial(
        pl.kernel,
        out_type=jax.ShapeDtypeStruct((nh, R * N, d), jnp.float32),
        mesh=_mesh(),
        scratch_types=[
            pltpu.VMEM_SHARED((N, d), jnp.float32),
            pltpu.VMEM_SHARED((N, d), jnp.float32),
            pltpu.VMEM((SUP, _CH), jnp.int32),
            pltpu.VMEM((SUP, _CH), jnp.int32),
            pltpu.VMEM((SUP, _CH, d), jnp.float32),
            pltpu.SemaphoreType.DMA((SUP,)),
            pltpu.SemaphoreType.DMA((SUP,)),
        ],
        compiler_params=pltpu.CompilerParams(use_tc_tiling_on_sc=False),
    )
    def k(t_hbm, src_hbm, dst_hbm, z_hbm, out_hbm, acc, hsp, sidx, didx,
          rows, gsem, ssem):
        c = lax.axis_index("c")
        s = lax.axis_index("s")
        drow0 = s * EPT // _CH
        for half in range(nh):
            for step in range(RPC):
                r = c * RPC + step

                @pl.when(s < _NS - 1)
                def _():
                    pltpu.sync_copy(
                        t_hbm.at[half, pl.ds(r * N + s * RPT, RPT)],
                        hsp.at[pl.ds(s * RPT, RPT)])
                    pltpu.sync_copy(z_hbm.at[pl.ds(0, RPT)],
                                    acc.at[pl.ds(s * RPT, RPT)])

                @pl.when(s == _NS - 1)
                def _():
                    pltpu.sync_copy(
                        t_hbm.at[half, pl.ds(r * N + (_NS - 1) * RPT, RLAST)],
                        hsp.at[pl.ds((_NS - 1) * RPT, RLAST)])
                    pltpu.sync_copy(z_hbm,
                                    acc.at[pl.ds((_NS - 1) * RPT, RLAST)])

                plsc.subcore_barrier()

                def process(drow, ns):
                    pltpu.sync_copy(src_hbm.at[pl.ds(drow, ns)],
                                    sidx.at[pl.ds(0, ns)])
                    pltpu.sync_copy(dst_hbm.at[pl.ds(drow, ns)],
                                    didx.at[pl.ds(0, ns)])
                    gd = [pltpu.async_copy(hsp.at[sidx.at[j]], rows.at[j],
                                           gsem.at[j])
                          for j in range(ns)]
                    sd = []
                    for j in range(ns):
                        gd[j].wait()
                        sd.append(
                            pltpu.async_copy(rows.at[j], acc.at[didx.at[j]],
                                             ssem.at[j], add=True))
                    for dsc in sd:
                        dsc.wait()

                def body(i, carry):
                    process(drow0 + i * SUP, SUP)
                    return carry

                lax.fori_loop(0, NIT, body, 0)
                if TAIL:
                    process(drow0 + NIT * SUP, TAIL)
                plsc.subcore_barrier()

                @pl.when(s < _NS - 1)
                def _():
                    pltpu.sync_copy(
                        acc.at[pl.ds(s * RPT, RPT)],
                        out_hbm.at[half, pl.ds(r * N + s * RPT, RPT)])

                @pl.when(s == _NS - 1)
                def _():
                    pltpu.sync_copy(
                        acc.at[pl.ds((_NS - 1) * RPT, RLAST)],
                        out_hbm.at[half,
                                   pl.ds(r * N + (_NS - 1) * RPT, RLAST)])

                plsc.subcore_barrier()

    return k(table, src, dst, zeros)


def _pool_sc_call(y, batch, zg, N, G):
    NCH = N // _CH
    NW = _NC * _NS
    PER = -(-NCH // NW)

    @functools.partial(
        pl.kernel,
        out_type=jax.ShapeDtypeStruct((_NC, G, _WD), jnp.float32),
        mesh=_mesh(),
        scratch_types=[
            pltpu.VMEM_SHARED((G, _WD), jnp.float32),
            pltpu.VMEM((_CH,), jnp.int32),
            pltpu.VMEM((_CH, _WD), jnp.float32),
        ],
        compiler_params=pltpu.CompilerParams(use_tc_tiling_on_sc=False),
    )
    def k(y_hbm, b_hbm, z_hbm, out_hbm, acc, bidx, rows):
        c = lax.axis_index("c")
        s = lax.axis_index("s")
        w = c * _NS + s

        @pl.when(s == 0)
        def _():
            pltpu.sync_copy(z_hbm, acc)

        plsc.subcore_barrier()
        for kk in range(PER):
            ch = kk * NW + w

            @pl.when(ch < NCH)
            def _():
                base = ch * _CH
                pltpu.sync_copy(b_hbm.at[pl.ds(base, _CH)], bidx)
                pltpu.sync_copy(y_hbm.at[pl.ds(base, _CH)], rows)
                pltpu.sync_copy(rows, acc.at[bidx], add=True)

        plsc.subcore_barrier()

        @pl.when(s == 0)
        def _():
            pltpu.sync_copy(acc, out_hbm.at[c])

    return k(y, batch, zg)


def _xr_call(x, keep3, R, N, F):
    BN = 400

    def body(x_ref, k_ref, o_ref):
        o_ref[...] = x_ref[...][None] * k_ref[...]

    return pl.pallas_call(
        body,
        grid=(R, N // BN),
        in_specs=[pl.BlockSpec((BN, F), lambda r, b: (b, 0)),
                  pl.BlockSpec((1, BN, 1), lambda r, b: (r, b, 0))],
        out_specs=pl.BlockSpec((1, BN, F), lambda r, b: (r, b, 0)),
        out_shape=jax.ShapeDtypeStruct((R, N, F), jnp.float32),
    )(x, keep3)


def _mlp_call(h, agg, nh, w1, b1, w2, b2, M, din, dout):
    BN = 400
    dh = din // nh

    def body(*refs):
        h_ref = refs[0]
        a_refs = refs[1:1 + nh]
        w1_ref, b1_ref, w2_ref, b2_ref, o_ref = refs[1 + nh:]
        if nh == 1:
            a = a_refs[0][0]
        else:
            a = jnp.concatenate([ar[0] for ar in a_refs], axis=-1)
        z = h_ref[...] + a
        z = jnp.dot(z, w1_ref[...], preferred_element_type=jnp.float32)
        z = jnp.maximum(z + b1_ref[...], 0.0)
        z = jnp.dot(z, w2_ref[...], preferred_element_type=jnp.float32)
        o_ref[...] = jnp.maximum(z + b2_ref[...], 0.0)

    in_specs = [pl.BlockSpec((BN, din), lambda b: (b, 0))]
    for half in range(nh):
        in_specs.append(
            pl.BlockSpec((1, BN, dh), lambda b, h=half: (h, b, 0)))
    in_specs += [pl.BlockSpec((din, dout), lambda b: (0, 0)),
                 pl.BlockSpec((1, dout), lambda b: (0, 0)),
                 pl.BlockSpec((dout, dout), lambda b: (0, 0)),
                 pl.BlockSpec((1, dout), lambda b: (0, 0))]
    return pl.pallas_call(
        body,
        grid=(M // BN,),
        in_specs=in_specs,
        out_specs=pl.BlockSpec((BN, dout), lambda b: (b, 0)),
        out_shape=jax.ShapeDtypeStruct((M, dout), jnp.float32),
    )(h, *([agg] * nh), w1, b1, w2, b2)


def _pernode_call(x, km, hs, w0, ws, N, F, D, R):
    BN = 400
    nh = len(hs)

    def body(*refs):
        x_ref, km_ref = refs[0], refs[1]
        h_refs = refs[2:2 + nh]
        w0_ref = refs[2 + nh]
        w_refs = refs[3 + nh:3 + 2 * nh]
        o_ref = refs[-1]
        y = jnp.dot(x_ref[...] * km_ref[...], w0_ref[...],
                    preferred_element_type=jnp.float32)
        for hr, wr in zip(h_refs, w_refs):
            m = hr[0]
            for r in range(1, R):
                m = m + hr[r]
            y = y + jnp.dot(m * (1.0 / R), wr[...],
                            preferred_element_type=jnp.float32)
        o_ref[...] = y

    in_specs = [pl.BlockSpec((BN, F), lambda b: (b, 0)),
                pl.BlockSpec((BN, 1), lambda b: (b, 0))]
    in_specs += [pl.BlockSpec((R, BN, D), lambda b: (0, b, 0))] * nh
    in_specs += [pl.BlockSpec((F, _WD), lambda b: (0, 0))]
    in_specs += [pl.BlockSpec((D, _WD), lambda b: (0, 0))] * nh
    return pl.pallas_call(
        body,
        grid=(N // BN,),
        in_specs=in_specs,
        out_specs=pl.BlockSpec((BN, _WD), lambda b: (b, 0)),
        out_shape=jax.ShapeDtypeStruct((N, _WD), jnp.float32),
    )(x, km, *hs, w0, *ws)


def _head_call(parts, bsum, G, C):
    def body(p_ref, b_ref, o_ref):
        p = p_ref[0] + p_ref[1] + b_ref[...]
        col = lax.broadcasted_iota(jnp.int32, (G, _WD), 1)
        valid = col < C
        pm = jnp.where(valid, p, jnp.full_like(p, -1e30))
        m = jnp.max(pm, axis=1, keepdims=True)
        e = jnp.where(valid, jnp.exp(p - m), 0.0)
        ssum = jnp.sum(e, axis=1, keepdims=True)
        o_ref[...] = p - m - jnp.log(ssum)

    return pl.pallas_call(
        body,
        in_specs=[pl.BlockSpec((2, G, _WD), lambda: (0, 0, 0)),
                  pl.BlockSpec((1, _WD), lambda: (0, 0))],
        out_specs=pl.BlockSpec((G, _WD), lambda: (0, 0)),
        out_shape=jax.ShapeDtypeStruct((G, _WD), jnp.float32),
    )(parts, bsum)


def kernel(x, edge_index, batch, params):
    N, F = x.shape
    E = edge_index.shape[1]
    D = params["conv0_w1"].shape[1]
    C = params["fc0_w"].shape[1]
    L = sum(1 for k in params if k.startswith("conv") and k.endswith("_w1"))
    R, G = _R, _G
    f32 = jnp.float32

    drop = jax.random.bernoulli(jax.random.key(42), _P, (R, N))
    keep = 1.0 - drop.astype(f32)
    keep3 = keep[:, :, None]
    km = jnp.mean(keep, axis=0)[:, None]

    src = edge_index[0].reshape(E // _CH, _CH)
    dst = edge_index[1].reshape(E // _CH, _CH)

    s_bn = 1.0 / jnp.sqrt(jnp.asarray(1.0 + 1e-5, f32))

    xr = _xr_call(x, keep3, R, N, F)
    h = xr.reshape(R * N, F)
    zeros = jnp.zeros((N - (_NS - 1) * ((N // _NS) // 8 * 8), D), f32)
    hs = []
    din = F
    for i in range(L):
        g1 = params[f"conv{i}_bn_g"] * s_bn
        w1 = params[f"conv{i}_w1"] * g1[None, :]
        b1 = (params[f"conv{i}_b1"] * g1 + params[f"conv{i}_bn_b"])[None, :]
        g2 = params[f"bn{i}_g"] * s_bn
        w2 = params[f"conv{i}_w2"] * g2[None, :]
        b2 = (params[f"conv{i}_b2"] * g2 + params[f"bn{i}_b"])[None, :]
        nh = din // D
        if nh == 1:
            table = h[None]
        else:
            table = (h.reshape(R * N, nh, D)
                     .transpose(1, 0, 2))
        agg = _seg_sum_call(table, src, dst, zeros, R, N, D, E, nh)
        h = _mlp_call(h, agg, nh, w1, b1, w2, b2, R * N, din, D)
        hs.append(h.reshape(R, N, D))
        din = D

    w0 = jnp.zeros((F, _WD), f32).at[:, :C].set(params["fc0_w"])
    ws = [jnp.zeros((D, _WD), f32).at[:, :C].set(params[f"fc{i + 1}_w"])
          for i in range(L)]
    bsum = sum(params[f"fc{i}_b"] for i in range(L + 1))
    bsum16 = jnp.zeros((1, _WD), f32).at[0, :C].set(bsum)

    y = _pernode_call(x, km, hs, w0, ws, N, F, D, R)
    zg = jnp.zeros((G, _WD), f32)
    parts = _pool_sc_call(y, batch, zg, N, G)
    out16 = _head_call(parts, bsum16, G, C)
    return out16[:, :C]


# speedup vs baseline: 1.0678x; 1.0483x over previous
import functools

import jax
import jax.numpy as jnp
from jax import lax
from jax.experimental import pallas as pl
from jax.experimental.pallas import tpu as pltpu
from jax.experimental.pallas import tpu_sc as plsc

_R = 4
_G = 64
_P = 0.1
_NC = 2
_NS = 16
_CH = 80
_WD = 16


def _mesh():
    return plsc.VectorSubcoreMesh(
        core_axis_name="c", subcore_axis_name="s",
        num_cores=_NC, num_subcores=_NS)


def _seg_sum_call(h, srcflat, dst, zeros, R, N, d, E):
    SUP = 4 if d >= 128 else 10
    EPT = E // _NS
    NIT = EPT // (SUP * _CH)
    TAIL = EPT // _CH - NIT * SUP
    RPC = R // _NC
    RPT = (N // _NS) // 8 * 8
    RLAST = N - (_NS - 1) * RPT

    @functools.partial(
        pl.kernel,
        out_type=jax.ShapeDtypeStruct((R * N, d), jnp.float32),
        mesh=_mesh(),
        scratch_types=[
            pltpu.VMEM_SHARED((N, d), jnp.float32),
            pltpu.VMEM((SUP, _CH), jnp.int32),
            pltpu.VMEM((SUP, _CH), jnp.int32),
            pltpu.VMEM((SUP, _CH, d), jnp.float32),
            pltpu.SemaphoreType.DMA((SUP,)),
            pltpu.SemaphoreType.DMA((SUP,)),
        ],
        compiler_params=pltpu.CompilerParams(use_tc_tiling_on_sc=False),
    )
    def k(h_hbm, src_hbm, dst_hbm, z_hbm, out_hbm, acc, sidx, didx, rows,
          gsem, ssem):
        c = lax.axis_index("c")
        s = lax.axis_index("s")
        for step in range(RPC):
            r = c * RPC + step

            @pl.when(s < _NS - 1)
            def _():
                pltpu.sync_copy(z_hbm.at[pl.ds(0, RPT)],
                                acc.at[pl.ds(s * RPT, RPT)])

            @pl.when(s == _NS - 1)
            def _():
                pltpu.sync_copy(z_hbm,
                                acc.at[pl.ds((_NS - 1) * RPT, RLAST)])

            plsc.subcore_barrier()

            srow0 = (r * E + s * EPT) // _CH
            drow0 = s * EPT // _CH

            def process(srow, drow, ns):
                pltpu.sync_copy(src_hbm.at[pl.ds(srow, ns)],
                                sidx.at[pl.ds(0, ns)])
                pltpu.sync_copy(dst_hbm.at[pl.ds(drow, ns)],
                                didx.at[pl.ds(0, ns)])
                gd = [pltpu.async_copy(h_hbm.at[sidx.at[j]], rows.at[j],
                                       gsem.at[j])
                      for j in range(ns)]
                sd = []
                for j in range(ns):
                    gd[j].wait()
                    sd.append(
                        pltpu.async_copy(rows.at[j], acc.at[didx.at[j]],
                                         ssem.at[j], add=True))
                for dsc in sd:
                    dsc.wait()

            def body(i, carry):
                process(srow0 + i * SUP, drow0 + i * SUP, SUP)
                return carry

            lax.fori_loop(0, NIT, body, 0)
            if TAIL:
                process(srow0 + NIT * SUP, drow0 + NIT * SUP, TAIL)
            plsc.subcore_barrier()

            @pl.when(s < _NS - 1)
            def _():
                pltpu.sync_copy(acc.at[pl.ds(s * RPT, RPT)],
                                out_hbm.at[pl.ds(r * N + s * RPT, RPT)])

            @pl.when(s == _NS - 1)
            def _():
                pltpu.sync_copy(
                    acc.at[pl.ds((_NS - 1) * RPT, RLAST)],
                    out_hbm.at[pl.ds(r * N + (_NS - 1) * RPT, RLAST)])

            plsc.subcore_barrier()

    return k(h, srcflat, dst, zeros)


def _pool_sc_call(y, batch, zg, N, G):
    NCH = N // _CH
    NW = _NC * _NS
    PER = -(-NCH // NW)

    @functools.partial(
        pl.kernel,
        out_type=jax.ShapeDtypeStruct((_NC, G, _WD), jnp.float32),
        mesh=_mesh(),
        scratch_types=[
            pltpu.VMEM_SHARED((G, _WD), jnp.float32),
            pltpu.VMEM((_CH,), jnp.int32),
            pltpu.VMEM((_CH, _WD), jnp.float32),
        ],
        compiler_params=pltpu.CompilerParams(use_tc_tiling_on_sc=False),
    )
    def k(y_hbm, b_hbm, z_hbm, out_hbm, acc, bidx, rows):
        c = lax.axis_index("c")
        s = lax.axis_index("s")
        w = c * _NS + s

        @pl.when(s == 0)
        def _():
            pltpu.sync_copy(z_hbm, acc)

        plsc.subcore_barrier()
        for kk in range(PER):
            ch = kk * NW + w

            @pl.when(ch < NCH)
            def _():
                base = ch * _CH
                pltpu.sync_copy(b_hbm.at[pl.ds(base, _CH)], bidx)
                pltpu.sync_copy(y_hbm.at[pl.ds(base, _CH)], rows)
                pltpu.sync_copy(rows, acc.at[bidx], add=True)

        plsc.subcore_barrier()

        @pl.when(s == 0)
        def _():
            pltpu.sync_copy(acc, out_hbm.at[c])

    return k(y, batch, zg)


def _xr_call(x, keep3, R, N, F):
    BN = 400

    def body(x_ref, k_ref, o_ref):
        o_ref[...] = x_ref[...][None] * k_ref[...]

    return pl.pallas_call(
        body,
        grid=(R, N // BN),
        in_specs=[pl.BlockSpec((BN, F), lambda r, b: (b, 0)),
                  pl.BlockSpec((1, BN, 1), lambda r, b: (r, b, 0))],
        out_specs=pl.BlockSpec((1, BN, F), lambda r, b: (r, b, 0)),
        out_shape=jax.ShapeDtypeStruct((R, N, F), jnp.float32),
    )(x, keep3)


def _mlp_call(h, agg, w1, b1, w2, b2, M, din, dout):
    BN = 400

    def body(h_ref, a_ref, w1_ref, b1_ref, w2_ref, b2_ref, o_ref):
        z = h_ref[...] + a_ref[...]
        z = jnp.dot(z, w1_ref[...], preferred_element_type=jnp.float32)
        z = jnp.maximum(z + b1_ref[...], 0.0)
        z = jnp.dot(z, w2_ref[...], preferred_element_type=jnp.float32)
        o_ref[...] = jnp.maximum(z + b2_ref[...], 0.0)

    return pl.pallas_call(
        body,
        grid=(M // BN,),
        in_specs=[pl.BlockSpec((BN, din), lambda b: (b, 0)),
                  pl.BlockSpec((BN, din), lambda b: (b, 0)),
                  pl.BlockSpec((din, dout), lambda b: (0, 0)),
                  pl.BlockSpec((1, dout), lambda b: (0, 0)),
                  pl.BlockSpec((dout, dout), lambda b: (0, 0)),
                  pl.BlockSpec((1, dout), lambda b: (0, 0))],
        out_specs=pl.BlockSpec((BN, dout), lambda b: (b, 0)),
        out_shape=jax.ShapeDtypeStruct((M, dout), jnp.float32),
    )(h, agg, w1, b1, w2, b2)


def _pernode_call(x, km, hs, w0, ws, N, F, D, R):
    BN = 400
    nh = len(hs)

    def body(*refs):
        x_ref, km_ref = refs[0], refs[1]
        h_refs = refs[2:2 + nh]
        w0_ref = refs[2 + nh]
        w_refs = refs[3 + nh:3 + 2 * nh]
        o_ref = refs[-1]
        y = jnp.dot(x_ref[...] * km_ref[...], w0_ref[...],
                    preferred_element_type=jnp.float32)
        for hr, wr in zip(h_refs, w_refs):
            m = hr[0]
            for r in range(1, R):
                m = m + hr[r]
            y = y + jnp.dot(m * (1.0 / R), wr[...],
                            preferred_element_type=jnp.float32)
        o_ref[...] = y

    in_specs = [pl.BlockSpec((BN, F), lambda b: (b, 0)),
                pl.BlockSpec((BN, 1), lambda b: (b, 0))]
    in_specs += [pl.BlockSpec((R, BN, D), lambda b: (0, b, 0))] * nh
    in_specs += [pl.BlockSpec((F, _WD), lambda b: (0, 0))]
    in_specs += [pl.BlockSpec((D, _WD), lambda b: (0, 0))] * nh
    return pl.pallas_call(
        body,
        grid=(N // BN,),
        in_specs=in_specs,
        out_specs=pl.BlockSpec((BN, _WD), lambda b: (b, 0)),
        out_shape=jax.ShapeDtypeStruct((N, _WD), jnp.float32),
    )(x, km, *hs, w0, *ws)


def _head_call(parts, bsum, G, C):
    def body(p_ref, b_ref, o_ref):
        p = p_ref[0] + p_ref[1] + b_ref[...]
        col = lax.broadcasted_iota(jnp.int32, (G, _WD), 1)
        valid = col < C
        pm = jnp.where(valid, p, jnp.full_like(p, -1e30))
        m = jnp.max(pm, axis=1, keepdims=True)
        e = jnp.where(valid, jnp.exp(p - m), 0.0)
        ssum = jnp.sum(e, axis=1, keepdims=True)
        o_ref[...] = p - m - jnp.log(ssum)

    return pl.pallas_call(
        body,
        in_specs=[pl.BlockSpec((2, G, _WD), lambda: (0, 0, 0)),
                  pl.BlockSpec((1, _WD), lambda: (0, 0))],
        out_specs=pl.BlockSpec((G, _WD), lambda: (0, 0)),
        out_shape=jax.ShapeDtypeStruct((G, _WD), jnp.float32),
    )(parts, bsum)


def kernel(x, edge_index, batch, params):
    N, F = x.shape
    E = edge_index.shape[1]
    D = params["conv0_w1"].shape[1]
    C = params["fc0_w"].shape[1]
    L = sum(1 for k in params if k.startswith("conv") and k.endswith("_w1"))
    R, G = _R, _G
    f32 = jnp.float32

    drop = jax.random.bernoulli(jax.random.key(42), _P, (R, N))
    keep = 1.0 - drop.astype(f32)
    keep3 = keep[:, :, None]
    km = jnp.mean(keep, axis=0)[:, None]

    src = edge_index[0]
    dst = edge_index[1].reshape(E // _CH, _CH)
    roffs = (jnp.arange(R, dtype=jnp.int32) * N)[:, None]
    srcflat = (src[None, :] + roffs).reshape(R * E // _CH, _CH)

    s_bn = 1.0 / jnp.sqrt(jnp.asarray(1.0 + 1e-5, f32))

    h = _xr_call(x, keep3, R, N, F).reshape(R * N, F)
    hs = []
    din = F
    for i in range(L):
        g1 = params[f"conv{i}_bn_g"] * s_bn
        w1 = params[f"conv{i}_w1"] * g1[None, :]
        b1 = (params[f"conv{i}_b1"] * g1 + params[f"conv{i}_bn_b"])[None, :]
        g2 = params[f"bn{i}_g"] * s_bn
        w2 = params[f"conv{i}_w2"] * g2[None, :]
        b2 = (params[f"conv{i}_b2"] * g2 + params[f"bn{i}_b"])[None, :]
        z = jnp.zeros((N - (_NS - 1) * ((N // _NS) // 8 * 8), din), f32)
        agg = _seg_sum_call(h, srcflat, dst, z, R, N, din, E)
        h = _mlp_call(h, agg, w1, b1, w2, b2, R * N, din, D)
        hs.append(h.reshape(R, N, D))
        din = D

    w0 = jnp.zeros((F, _WD), f32).at[:, :C].set(params["fc0_w"])
    ws = [jnp.zeros((D, _WD), f32).at[:, :C].set(params[f"fc{i + 1}_w"])
          for i in range(L)]
    bsum = sum(params[f"fc{i}_b"] for i in range(L + 1))
    bsum16 = jnp.zeros((1, _WD), f32).at[0, :C].set(bsum)

    y = _pernode_call(x, km, hs, w0, ws, N, F, D, R)
    zg = jnp.zeros((G, _WD), f32)
    parts = _pool_sc_call(y, batch, zg, N, G)
    out16 = _head_call(parts, bsum16, G, C)
    return out16[:, :C]
